# baseline (speedup 1.0000x reference)
import functools

import jax
import jax.numpy as jnp
from jax import lax
from jax.experimental import pallas as pl
from jax.experimental.pallas import tpu as pltpu
from jax.experimental.pallas import tpu_sc as plsc

N_ROWS = 1_000_000
N_COLS = 10
N_BINS = 15
NC, NS, L = 2, 16, 16
NW = NC * NS
CHUNK_ROWS = 1000
CHUNK_WORDS = CHUNK_ROWS * N_COLS
N_CHUNKS = N_ROWS // CHUNK_ROWS
GROUPS = (CHUNK_ROWS + L - 1) // L
BUF_WORDS = GROUPS * L * N_COLS


def _ece_partials(logits_flat, labels_flat):
    mesh = plsc.VectorSubcoreMesh(
        core_axis_name="c", subcore_axis_name="s",
        num_cores=NC, num_subcores=NS)

    @functools.partial(
        pl.kernel,
        out_type=jax.ShapeDtypeStruct((NW, 48, L), jnp.float32),
        mesh=mesh,
        scratch_types=[
            pltpu.VMEM((BUF_WORDS,), jnp.float32),
            pltpu.VMEM((BUF_WORDS,), jnp.float32),
            pltpu.VMEM((48, L), jnp.float32),
        ],
        compiler_params=pltpu.CompilerParams(needs_layout_passes=False),
    )
    def sc_kernel(logits_hbm, labels_hbm, out_hbm, lbuf, bbuf, part):
        wid = lax.axis_index("s") * NC + lax.axis_index("c")

        zeros16 = jnp.zeros((L,), jnp.float32)
        for r in range(48):
            part[r, :] = zeros16

        lane = lax.broadcasted_iota(jnp.int32, (L,), 0)
        row_base = lane * N_COLS
        ones_f = jnp.full((L,), 1.0, jnp.float32)
        fifteen = jnp.full((L,), float(N_BINS), jnp.float32)

        def do_chunk(c, _):
            off = pl.multiple_of(c * CHUNK_WORDS, 8)
            pltpu.sync_copy(logits_hbm.at[pl.ds(off, CHUNK_WORDS)],
                            lbuf.at[pl.ds(0, CHUNK_WORDS)])
            pltpu.sync_copy(labels_hbm.at[pl.ds(off, CHUNK_WORDS)],
                            bbuf.at[pl.ds(0, CHUNK_WORDS)])

            def do_group(g, _):
                base = row_base + g * (L * N_COLS)
                l0 = plsc.load_gather(lbuf, [base])
                m = l0
                still_neg = l0 < 0.0
                lead = jnp.where(still_neg, 1, 0).astype(jnp.int32)
                for j in range(1, N_COLS):
                    lj = plsc.load_gather(lbuf, [base + j])
                    m = jnp.maximum(m, lj)
                    still_neg = jnp.logical_and(still_neg, lj < 0.0)
                    lead = lead + jnp.where(still_neg, 1, 0).astype(jnp.int32)
                pred_idx = jnp.where(lead == N_COLS, 0, lead)

                b0 = plsc.load_gather(bbuf, [base])
                best = b0
                lidx = jnp.zeros((L,), jnp.int32)
                for j in range(1, N_COLS):
                    bj = plsc.load_gather(bbuf, [base + j])
                    gt = bj > best
                    best = jnp.maximum(best, bj)
                    lidx = jnp.where(gt, j, lidx)

                acc = jnp.where(pred_idx == lidx, 1.0, 0.0).astype(jnp.float32)
                conf = ones_f / (ones_f + jnp.exp(-m))
                bin_i = (conf * fifteen).astype(jnp.int32)
                bin_i = jnp.minimum(jnp.maximum(bin_i, 0), N_BINS - 1)

                row_id = g * L + lane
                valid = jnp.logical_and(row_id < CHUNK_ROWS, conf > 0.0)
                r0 = bin_i
                plsc.addupdate_scatter(part, [r0, lane], ones_f, mask=valid)
                plsc.addupdate_scatter(part, [r0 + 16, lane], acc, mask=valid)
                plsc.addupdate_scatter(part, [r0 + 32, lane], conf, mask=valid)
                return 0

            lax.fori_loop(0, GROUPS, do_group, 0)
            return 0

        n_my = (N_CHUNKS - wid + NW - 1) // NW

        def chunk_iter(k, _):
            return do_chunk(wid + k * NW, _)

        lax.fori_loop(0, n_my, chunk_iter, 0)
        pltpu.sync_copy(part, out_hbm.at[wid])

    return sc_kernel(logits_flat, labels_flat)


def _combine(partials):
    def tc_kernel(p_ref, o_ref):
        x = p_ref[...]
        tot = jnp.sum(x, axis=(0, 2))
        cnt = tot[0:16]
        acc_s = tot[16:32]
        conf_s = tot[32:48]
        prop = cnt * (1.0 / N_ROWS)
        safe = jnp.maximum(cnt, 1.0)
        contrib = jnp.abs(conf_s / safe - acc_s / safe) * prop
        contrib = jnp.where(cnt > 0.0, contrib, 0.0)
        o_ref[0, 0] = jnp.sum(contrib)

    out = pl.pallas_call(
        tc_kernel,
        out_shape=jax.ShapeDtypeStruct((1, 1), jnp.float32),
        in_specs=[pl.BlockSpec(memory_space=pltpu.VMEM)],
        out_specs=pl.BlockSpec(memory_space=pltpu.SMEM),
    )(partials)
    return out.reshape((1,))


@jax.jit
def kernel(logits, labels):
    partials = _ece_partials(logits.reshape(-1), labels.reshape(-1))
    return _combine(partials)


# speedup vs baseline: 1.2755x; 1.2755x over previous
import functools

import jax
import jax.numpy as jnp
from jax import lax
from jax.experimental import pallas as pl
from jax.experimental.pallas import tpu as pltpu
from jax.experimental.pallas import tpu_sc as plsc

N_ROWS = 1_000_000
N_COLS = 10
N_BINS = 15
NC, NS, L = 2, 16, 16
NW = NC * NS
CHUNK_ROWS = 160
N_CHUNKS = N_ROWS // CHUNK_ROWS
GROUPS = CHUNK_ROWS // L


def _ece_partials(logits2d, labels2d):
    mesh = plsc.VectorSubcoreMesh(
        core_axis_name="c", subcore_axis_name="s",
        num_cores=NC, num_subcores=NS)

    buf_t = pltpu.VMEM((CHUNK_ROWS, N_COLS), jnp.float32)

    @functools.partial(
        pl.kernel,
        out_type=jax.ShapeDtypeStruct((NW, 48, L), jnp.float32),
        mesh=mesh,
        scratch_types=[
            buf_t, buf_t,
            buf_t, buf_t,
            pltpu.VMEM((48, L), jnp.float32),
            pltpu.SemaphoreType.DMA,
            pltpu.SemaphoreType.DMA,
            pltpu.SemaphoreType.DMA,
            pltpu.SemaphoreType.DMA,
        ],
        compiler_params=pltpu.CompilerParams(
            needs_layout_passes=False, use_tc_tiling_on_sc=True),
    )
    def sc_kernel(logits_hbm, labels_hbm, out_hbm,
                  lbuf0, lbuf1, bbuf0, bbuf1, part,
                  sl0, sl1, sb0, sb1):
        wid = lax.axis_index("s") * NC + lax.axis_index("c")

        zeros16 = jnp.zeros((L,), jnp.float32)
        for r in range(48):
            part[r, :] = zeros16

        lane = lax.broadcasted_iota(jnp.int32, (L,), 0)
        ones_f = jnp.full((L,), 1.0, jnp.float32)
        fifteen = jnp.full((L,), float(N_BINS), jnp.float32)

        def issue(c, lb, bb, sl, sb):
            off = pl.multiple_of(c * CHUNK_ROWS, 8)
            pltpu.make_async_copy(
                logits_hbm.at[pl.ds(off, CHUNK_ROWS), :], lb, sl).start()
            pltpu.make_async_copy(
                labels_hbm.at[pl.ds(off, CHUNK_ROWS), :], bb, sb).start()

        def wait(lb, bb, sl, sb):
            pltpu.make_async_copy(
                logits_hbm.at[pl.ds(0, CHUNK_ROWS), :], lb, sl).wait()
            pltpu.make_async_copy(
                labels_hbm.at[pl.ds(0, CHUNK_ROWS), :], bb, sb).wait()

        def compute(lb, bb):
            def do_group(g, _):
                rows = lane + g * L
                col0 = jnp.zeros((L,), jnp.int32)
                l0 = plsc.load_gather(lb, [rows, col0])
                m = l0
                still_neg = l0 < 0.0
                lead = jnp.where(still_neg, 1, 0).astype(jnp.int32)
                for j in range(1, N_COLS):
                    lj = plsc.load_gather(lb, [rows, col0 + j])
                    m = jnp.maximum(m, lj)
                    still_neg = jnp.logical_and(still_neg, lj < 0.0)
                    lead = lead + jnp.where(still_neg, 1, 0).astype(jnp.int32)
                pred_idx = jnp.where(lead == N_COLS, 0, lead)

                b0 = plsc.load_gather(bb, [rows, col0])
                best = b0
                lidx = jnp.zeros((L,), jnp.int32)
                for j in range(1, N_COLS):
                    bj = plsc.load_gather(bb, [rows, col0 + j])
                    gt = bj > best
                    best = jnp.maximum(best, bj)
                    lidx = jnp.where(gt, j, lidx)

                acc = jnp.where(pred_idx == lidx, 1.0, 0.0).astype(jnp.float32)
                conf = ones_f / (ones_f + jnp.exp(-m))
                bin_i = (conf * fifteen).astype(jnp.int32)
                bin_i = jnp.minimum(jnp.maximum(bin_i, 0), N_BINS - 1)

                valid = conf > 0.0
                plsc.addupdate_scatter(part, [bin_i, lane], ones_f,
                                       mask=valid)
                plsc.addupdate_scatter(part, [bin_i + 16, lane], acc,
                                       mask=valid)
                plsc.addupdate_scatter(part, [bin_i + 32, lane], conf,
                                       mask=valid)
                return 0

            lax.fori_loop(0, GROUPS, do_group, 0)

        nk = (N_CHUNKS - wid + NW - 1) // NW
        issue(wid, lbuf0, bbuf0, sl0, sb0)

        def body(k, _):
            nxt = wid + (k + 1) * NW
            even = (k % 2) == 0
            has_next = nxt < N_CHUNKS

            @pl.when(jnp.logical_and(has_next, even))
            def _():
                issue(nxt, lbuf1, bbuf1, sl1, sb1)

            @pl.when(jnp.logical_and(has_next, jnp.logical_not(even)))
            def _():
                issue(nxt, lbuf0, bbuf0, sl0, sb0)

            @pl.when(even)
            def _():
                wait(lbuf0, bbuf0, sl0, sb0)
                compute(lbuf0, bbuf0)

            @pl.when(jnp.logical_not(even))
            def _():
                wait(lbuf1, bbuf1, sl1, sb1)
                compute(lbuf1, bbuf1)

            return 0

        lax.fori_loop(0, nk, body, 0)
        pltpu.sync_copy(part, out_hbm.at[wid])

    return sc_kernel(logits2d, labels2d)


def _combine(partials):
    def tc_kernel(p_ref, o_ref):
        x = p_ref[...]
        tot = jnp.sum(x, axis=(0, 2))
        cnt = tot[0:16]
        acc_s = tot[16:32]
        conf_s = tot[32:48]
        prop = cnt * (1.0 / N_ROWS)
        safe = jnp.maximum(cnt, 1.0)
        contrib = jnp.abs(conf_s / safe - acc_s / safe) * prop
        contrib = jnp.where(cnt > 0.0, contrib, 0.0)
        o_ref[0, 0] = jnp.sum(contrib)

    out = pl.pallas_call(
        tc_kernel,
        out_shape=jax.ShapeDtypeStruct((1, 1), jnp.float32),
        in_specs=[pl.BlockSpec(memory_space=pltpu.VMEM)],
        out_specs=pl.BlockSpec(memory_space=pltpu.SMEM),
    )(partials)
    return out.reshape((1,))


@jax.jit
def kernel(logits, labels):
    partials = _ece_partials(logits, labels)
    return _combine(partials)
